# Initial kernel scaffold; baseline (speedup 1.0000x reference)
import jax
import jax.numpy as jnp
from jax.experimental import pallas as pl


def kernel(preds, margin):
    raise NotImplementedError("write your pallas kernel here")


# speedup vs baseline: 1.0479x; 1.0479x over previous
import jax
import jax.numpy as jnp
from jax.experimental import pallas as pl
from jax.experimental.pallas import tpu as pltpu

_N = 1048576
_ROWS = _N // 128
_BLOCK = 1024
_STEPS = _ROWS // _BLOCK


def _tc_body(p_ref, m_ref, o_ref, acc_ref):
    i = pl.program_id(0)
    m = m_ref[...]
    p = p_ref[...]
    contrib = jnp.where(m != 0.0, jnp.exp(-0.5 * m * m) * p, 0.0)
    part = jnp.sum(contrib.reshape(_BLOCK // 8, 8, 128), axis=0)

    @pl.when(i == 0)
    def _():
        acc_ref[...] = part

    @pl.when(i > 0)
    def _():
        acc_ref[...] += part

    @pl.when(i == _STEPS - 1)
    def _():
        o_ref[0, 0] = jnp.sum(acc_ref[...]) * (-1.0 / _N)


def kernel(preds, margin):
    p2 = preds.reshape(_ROWS, 128)
    m2 = margin.reshape(_ROWS, 128)
    out = pl.pallas_call(
        _tc_body,
        grid=(_STEPS,),
        in_specs=[
            pl.BlockSpec((_BLOCK, 128), lambda i: (i, 0)),
            pl.BlockSpec((_BLOCK, 128), lambda i: (i, 0)),
        ],
        out_specs=pl.BlockSpec(memory_space=pltpu.SMEM),
        out_shape=jax.ShapeDtypeStruct((1, 1), jnp.float32),
        scratch_shapes=[pltpu.VMEM((8, 128), jnp.float32)],
    )(p2, m2)
    return out[0, 0]


# speedup vs baseline: 1.0496x; 1.0016x over previous
import jax
import jax.numpy as jnp
from jax.experimental import pallas as pl
from jax.experimental.pallas import tpu as pltpu

_N = 1048576
_ROWS = _N // 128
_BLOCK = 1024
_STEPS = _ROWS // _BLOCK


def _tc_body(p_ref, m_ref, o_ref, acc_ref):
    i = pl.program_id(0)
    part = None
    for k in range(0, _BLOCK, 64):
        m = m_ref[pl.ds(k, 64), :]
        p = p_ref[pl.ds(k, 64), :]
        pm = jnp.where(m != 0.0, p, 0.0)
        c = jnp.exp(-0.5 * m * m) * pm
        part = c if part is None else part + c

    @pl.when(i == 0)
    def _():
        acc_ref[...] = part

    @pl.when(i > 0)
    def _():
        acc_ref[...] += part

    @pl.when(i == _STEPS - 1)
    def _():
        o_ref[0, 0] = jnp.sum(acc_ref[...]) * (-1.0 / _N)


def kernel(preds, margin):
    p2 = preds.reshape(_ROWS, 128)
    m2 = margin.reshape(_ROWS, 128)
    out = pl.pallas_call(
        _tc_body,
        grid=(_STEPS,),
        in_specs=[
            pl.BlockSpec((_BLOCK, 128), lambda i: (i, 0)),
            pl.BlockSpec((_BLOCK, 128), lambda i: (i, 0)),
        ],
        out_specs=pl.BlockSpec(memory_space=pltpu.SMEM),
        out_shape=jax.ShapeDtypeStruct((1, 1), jnp.float32),
        scratch_shapes=[pltpu.VMEM((64, 128), jnp.float32)],
    )(p2, m2)
    return out[0, 0]
